# Initial kernel scaffold; baseline (speedup 1.0000x reference)
import jax
import jax.numpy as jnp
from jax.experimental import pallas as pl


def kernel(x, edge_index, edge_attr, batch, params):
    raise NotImplementedError("write your pallas kernel here")


# speedup vs baseline: 1.9155x; 1.9155x over previous
import functools

import jax
import jax.numpy as jnp
from jax import lax
from jax.experimental import pallas as pl
from jax.experimental.pallas import tpu as pltpu
from jax.experimental.pallas import tpu_sc as plsc

N = 10000
E = 320000
D = 128
HD = D // 2
G = 64
L = 3

NCORE = 2
NSUB = 16
EPT = E // NSUB
C = 80
NCH = EPT // C
ZR = 632
ZTAIL = N - (NSUB - 1) * ZR


def _sc_aggregate(hin2, src3, dst3, cid3, ctab2, zeros):
  mesh = plsc.VectorSubcoreMesh(core_axis_name="c", subcore_axis_name="s")

  @functools.partial(
      pl.kernel,
      out_type=jax.ShapeDtypeStruct((NCORE, N, HD), jnp.float32),
      mesh=mesh,
      compiler_params=pltpu.CompilerParams(use_tc_tiling_on_sc=False),
      scratch_types=[
          pltpu.VMEM((NCH, C), jnp.int32),
          pltpu.VMEM((NCH, C), jnp.int32),
          pltpu.VMEM((C, HD), jnp.float32),
          pltpu.VMEM((C, HD), jnp.float32),
          pltpu.VMEM((C,), jnp.int32),
          pltpu.VMEM_SHARED((N, HD), jnp.float32),
          pltpu.SemaphoreType.DMA,
          pltpu.SemaphoreType.DMA,
      ],
  )
  def k(hin_h, src_h, dst_h, cid_h, ctab_h, zero_h, out_h,
        srcv, cidv, msg, cbuf, dchunk, aggr, sem1, sem2):
    c = lax.axis_index("c")
    s = lax.axis_index("s")

    @pl.when(s < NSUB - 1)
    def _():
      pltpu.sync_copy(zero_h.at[pl.ds(s * ZR, ZR)],
                      aggr.at[pl.ds(s * ZR, ZR)])

    @pl.when(s == NSUB - 1)
    def _():
      pltpu.sync_copy(zero_h.at[pl.ds((NSUB - 1) * ZR, ZTAIL)],
                      aggr.at[pl.ds((NSUB - 1) * ZR, ZTAIL)])

    pltpu.sync_copy(src_h.at[c, s], srcv)
    pltpu.sync_copy(cid_h.at[c, s], cidv)
    plsc.subcore_barrier()

    def chunk(kk, carry):
      pltpu.async_copy(hin_h.at[srcv.at[kk]], msg, sem1).wait()
      pltpu.async_copy(ctab_h.at[cidv.at[kk]], cbuf, sem2).wait()
      pltpu.sync_copy(dst_h.at[s, kk], dchunk)

      def edge(e, cc):
        for d in range(HD // 16):
          sl = pl.ds(d * 16, 16)
          msg[e, sl] = jnp.maximum(msg[e, sl] + cbuf[e, sl], 0.0)
        return cc

      lax.fori_loop(0, C, edge, 0, unroll=2)
      pltpu.sync_copy(msg, aggr.at[dchunk], add=True)
      return carry

    lax.fori_loop(0, NCH, chunk, 0)
    plsc.subcore_barrier()

    @pl.when(s < NSUB - 1)
    def _():
      pltpu.sync_copy(aggr.at[pl.ds(s * ZR, ZR)],
                      out_h.at[c, pl.ds(s * ZR, ZR)])

    @pl.when(s == NSUB - 1)
    def _():
      pltpu.sync_copy(aggr.at[pl.ds((NSUB - 1) * ZR, ZTAIL)],
                      out_h.at[c, pl.ds((NSUB - 1) * ZR, ZTAIL)])

  return k(hin2, src3, dst3, cid3, ctab2, zeros)


def _split2(h):
  return jnp.concatenate([h[:, :HD], h[:, HD:]], axis=0)


def _tc_hin0(x, vne_row):
  def body(x_ref, v_ref, o_ref):
    o_ref[...] = _split2(x_ref[...] + v_ref[...])

  return pl.pallas_call(
      body,
      out_shape=jax.ShapeDtypeStruct((2 * N, HD), jnp.float32),
  )(x, vne_row)


def _bn(t, g, b):
  m = jnp.mean(t, axis=0, keepdims=True)
  v = jnp.mean((t - m) * (t - m), axis=0, keepdims=True)
  return (t - m) / jnp.sqrt(v + 1e-5) * g + b


def _tc_node_mlp(hin2, ag, eps, W1, b1, g1, be1, W2, b2, gl, bl, relu_out):
  def body(hin_ref, ag_ref, eps_ref, W1_ref, b1_ref, g1_ref, be1_ref,
           W2_ref, b2_ref, gl_ref, bl_ref, ho_ref):
    e1 = 1.0 + eps_ref[0, 0]
    hin2v = hin_ref[...]
    t = jnp.concatenate([e1 * hin2v[:N] + ag_ref[0],
                         e1 * hin2v[N:] + ag_ref[1]], axis=1)
    u = jnp.dot(t.astype(jnp.bfloat16), W1_ref[...].astype(jnp.bfloat16), preferred_element_type=jnp.float32) + b1_ref[...]
    u = jnp.maximum(_bn(u, g1_ref[...], be1_ref[...]), 0.0)
    h = jnp.dot(u.astype(jnp.bfloat16), W2_ref[...].astype(jnp.bfloat16), preferred_element_type=jnp.float32) + b2_ref[...]
    h = _bn(h, gl_ref[...], bl_ref[...])
    if relu_out:
      h = jnp.maximum(h, 0.0)
    ho_ref[...] = h

  return pl.pallas_call(
      body,
      out_shape=jax.ShapeDtypeStruct((N, D), jnp.float32),
      compiler_params=pltpu.CompilerParams(vmem_limit_bytes=100 * 1024 * 1024),
  )(hin2, ag, eps, W1, b1, g1, be1, W2, b2, gl, bl)


def _tc_vn_update(hin2, vne, batch_row, ng,
                  mW1, mb1, mg1, mbe1, mW2, mb2, mg2, mbe2):
  def body(hin_ref, vne_ref, br_ref, ng_ref,
           mW1_ref, mb1_ref, mg1_ref, mbe1_ref,
           mW2_ref, mb2_ref, mg2_ref, mbe2_ref, vnext_ref):
    onehot_t = (lax.broadcasted_iota(jnp.int32, (G, N), 0)
                == br_ref[...]).astype(jnp.float32)
    hin2v = hin_ref[...]
    s0 = jnp.dot(onehot_t, hin2v[:N], preferred_element_type=jnp.float32, precision=lax.Precision.HIGHEST)
    s1 = jnp.dot(onehot_t, hin2v[N:], preferred_element_type=jnp.float32, precision=lax.Precision.HIGHEST)
    vtemp = jnp.concatenate([s0, s1], axis=1) + vne_ref[...]

    ng = ng_ref[0, 0]
    nf = ng.astype(jnp.float32)
    maskcol = lax.broadcasted_iota(jnp.int32, (G, 1), 0) < ng

    def bn_masked(tt, g, b):
      tm = jnp.where(maskcol, tt, 0.0)
      m = jnp.sum(tm, axis=0, keepdims=True) / nf
      dlt = jnp.where(maskcol, tt - m, 0.0)
      v = jnp.sum(dlt * dlt, axis=0, keepdims=True) / nf
      return (tt - m) / jnp.sqrt(v + 1e-5) * g + b

    v1 = jnp.dot(vtemp.astype(jnp.bfloat16), mW1_ref[...].astype(jnp.bfloat16), preferred_element_type=jnp.float32) + mb1_ref[...]
    v1 = jnp.maximum(bn_masked(v1, mg1_ref[...], mbe1_ref[...]), 0.0)
    v2 = jnp.dot(v1.astype(jnp.bfloat16), mW2_ref[...].astype(jnp.bfloat16), preferred_element_type=jnp.float32) + mb2_ref[...]
    v2 = jnp.maximum(bn_masked(v2, mg2_ref[...], mbe2_ref[...]), 0.0)
    vnext_ref[...] = v2

  return pl.pallas_call(
      body,
      out_shape=jax.ShapeDtypeStruct((G, D), jnp.float32),
      compiler_params=pltpu.CompilerParams(vmem_limit_bytes=100 * 1024 * 1024),
  )(hin2, vne, batch_row, ng, mW1, mb1, mg1, mbe1, mW2, mb2, mg2, mbe2)


def _tc_hnext(h, v2, batch_row):
  def body(h_ref, v2_ref, br_ref, o_ref):
    onehot_t = (lax.broadcasted_iota(jnp.int32, (G, N), 0)
                == br_ref[...]).astype(jnp.float32)
    hadd = lax.dot_general(onehot_t, v2_ref[...],
                           dimension_numbers=(((0,), (0,)), ((), ())),
                           preferred_element_type=jnp.float32, precision=lax.Precision.HIGHEST)
    o_ref[...] = _split2(h_ref[...] + hadd)

  return pl.pallas_call(
      body,
      out_shape=jax.ShapeDtypeStruct((2 * N, HD), jnp.float32),
      compiler_params=pltpu.CompilerParams(vmem_limit_bytes=100 * 1024 * 1024),
  )(h, v2, batch_row)


def kernel(x, edge_index, edge_attr, batch, params):
  dstf = edge_index[1]
  order = jnp.argsort(dstf)
  jj = jnp.arange(E, dtype=jnp.int32)
  cht = NSUB * NCH
  chunk_id = jj % cht
  slot = jj // cht
  pslot = (41 * slot) % C
  flat = ((chunk_id // NCH) * (NCH * C) + (chunk_id % NCH) * C + pslot)
  inv = jnp.zeros((E,), jnp.int32).at[flat].set(order.astype(jnp.int32))
  src = edge_index[0][inv].reshape(NSUB, NCH, C)
  dst3 = dstf[inv].reshape(NSUB, NCH, C)
  cidf = (edge_attr[:, 0] * 64 + edge_attr[:, 1] * 8
          + edge_attr[:, 2]).astype(jnp.int32)
  cid = cidf[inv].reshape(NSUB, NCH, C)
  src3 = jnp.stack([src, src + N])
  cid3 = jnp.stack([cid, cid + 512])
  zeros = jnp.zeros((N, HD), jnp.float32)
  batch_row = batch.reshape(1, N)
  ng = (batch[-1] + 1).reshape(1, 1)

  def r2(a):
    return a.reshape(1, -1)

  vne = jnp.broadcast_to(params['vne_w'], (G, D))
  hin2 = _tc_hin0(x, params['vne_w'])

  for l in range(L):
    cv = params['convs'][l]
    ctab = (cv['bond_tables'][0][:, None, None, :]
            + cv['bond_tables'][1][None, :, None, :]
            + cv['bond_tables'][2][None, None, :, :]).reshape(512, D)
    ctab2 = _split2(ctab)
    ag = _sc_aggregate(hin2, src3, dst3, cid3, ctab2, zeros)
    eps = cv['eps'].reshape(1, 1)
    bn = params['bns'][l]
    h = _tc_node_mlp(hin2, ag, eps,
                     cv['W1'], r2(cv['b1']), r2(cv['bn1_g']), r2(cv['bn1_b']),
                     cv['W2'], r2(cv['b2']), r2(bn['g']), r2(bn['b']),
                     relu_out=(l != L - 1))
    if l == L - 1:
      return h
    m = params['vn_mlps'][l]
    v2 = _tc_vn_update(hin2, vne, batch_row, ng,
                       m['W1'], r2(m['b1']), r2(m['bn1_g']), r2(m['bn1_b']),
                       m['W2'], r2(m['b2']), r2(m['bn2_g']), r2(m['bn2_b']))
    hin2 = _tc_hnext(h, v2, batch_row)
    vne = v2
